# baseline (speedup 1.0000x reference)
import functools

import jax
import jax.numpy as jnp
from jax import lax
from jax.experimental import pallas as pl
from jax.experimental.pallas import tpu as pltpu
from jax.experimental.pallas import tpu_sc as plsc

V_DIM = 1000000
EMB_DIM = 64
BATCH = 16384

NC = 2
NS = 16
NW = NC * NS
B_PER_W = BATCH // NW
CHUNK = 128
NCHUNK = B_PER_W // CHUNK


def _emb_kernel(idx_hbm, table_hbm, out_hbm, idx_v, rows_v, sems):
    wid = lax.axis_index("s") * NC + lax.axis_index("c")
    base = wid * B_PER_W
    pltpu.sync_copy(idx_hbm.at[wid], idx_v)
    copies = []
    for j in range(NCHUNK):
        copies.append(
            pltpu.async_copy(
                table_hbm.at[idx_v.at[j]],
                rows_v.at[pl.ds(j * CHUNK, CHUNK)],
                sems.at[j],
            )
        )
    for c in copies:
        c.wait()
    pltpu.sync_copy(rows_v, out_hbm.at[pl.ds(base, B_PER_W)])


@jax.jit
def _emb_lookup(idx3, table):
    mesh = plsc.VectorSubcoreMesh(
        core_axis_name="c", subcore_axis_name="s", num_cores=NC, num_subcores=NS
    )
    return pl.kernel(
        _emb_kernel,
        out_type=jax.ShapeDtypeStruct((BATCH, EMB_DIM), jnp.float32),
        mesh=mesh,
        scratch_types=[
            pltpu.VMEM((NCHUNK, CHUNK), jnp.int32),
            pltpu.VMEM((B_PER_W, EMB_DIM), jnp.float32),
            pltpu.SemaphoreType.DMA((NCHUNK,)),
        ],
        compiler_params=pltpu.CompilerParams(use_tc_tiling_on_sc=False),
    )(idx3, table)


def kernel(x, embeddings_weight):
    idx3 = x.astype(jnp.int32).reshape(NW, NCHUNK, CHUNK)
    return _emb_lookup(idx3, embeddings_weight)


# speedup vs baseline: 2.5706x; 2.5706x over previous
import functools

import jax
import jax.numpy as jnp
from jax import lax
from jax.experimental import pallas as pl
from jax.experimental.pallas import tpu as pltpu
from jax.experimental.pallas import tpu_sc as plsc

V_DIM = 1000000
EMB_DIM = 64
BATCH = 16384

NC = 2
NS = 16
NW = NC * NS
B_PER_W = BATCH // NW
NBUF = 4
L = 16
NG = B_PER_W // L


def _emb_kernel(idx_hbm, tableT_hbm, out_hbm, idx_v, cols_v, outS_v, sems):
    wid = lax.axis_index("s") * NC + lax.axis_index("c")
    base = wid * B_PER_W
    pltpu.sync_copy(idx_hbm.at[wid], idx_v.at[pl.ds(0, B_PER_W)])

    def fire(x, slot):
        tc = lax.shift_right_logical(x, 7)
        off = pl.multiple_of(tc * 128, 128)
        pltpu.async_copy(
            tableT_hbm.at[:, pl.ds(off, 128)], cols_v.at[slot], sems.at[slot]
        )

    def wait(slot):
        pltpu.make_async_copy(
            tableT_hbm.at[:, pl.ds(0, 128)], cols_v.at[slot], sems.at[slot]
        ).wait()

    def select(x, i, slot):
        c = jnp.broadcast_to(jnp.bitwise_and(x, 127), (L,))
        for g in range(EMB_DIM // L):
            idx0 = lax.iota(jnp.int32, L) + g * L
            outS_v[i, pl.ds(g * L, L)] = plsc.load_gather(
                cols_v.at[slot], [idx0, c]
            )

    v0 = idx_v[pl.ds(0, L)]
    for b in range(NBUF):
        fire(v0[b], b)

    def body(g, _):
        vcur = idx_v[pl.ds(g * L, L)]
        vnxt = idx_v[pl.ds(g * L + L, L)]
        for b in range(L):
            i = g * L + b
            slot = b % NBUF
            wait(slot)
            xn = vcur[b + NBUF] if b + NBUF < L else vnxt[b + NBUF - L]

            @pl.when(i < B_PER_W - NBUF)
            def _():
                fire(xn, slot)

            select(vcur[b], i, slot)
        return _

    lax.fori_loop(0, NG, body, None, unroll=False)
    pltpu.sync_copy(outS_v, out_hbm.at[pl.ds(base, B_PER_W)])


@jax.jit
def _emb_lookup(idx2, tableT):
    mesh = plsc.VectorSubcoreMesh(
        core_axis_name="c", subcore_axis_name="s", num_cores=NC, num_subcores=NS
    )
    return pl.kernel(
        _emb_kernel,
        out_type=jax.ShapeDtypeStruct((BATCH, EMB_DIM), jnp.float32),
        mesh=mesh,
        scratch_types=[
            pltpu.VMEM((B_PER_W + L,), jnp.int32),
            pltpu.VMEM((NBUF, EMB_DIM, 128), jnp.float32),
            pltpu.VMEM((B_PER_W, EMB_DIM), jnp.float32),
            pltpu.SemaphoreType.DMA((NBUF,)),
        ],
        compiler_params=pltpu.CompilerParams(needs_layout_passes=False),
    )(idx2, tableT)


def kernel(x, embeddings_weight):
    idx2 = x.astype(jnp.int32).reshape(NW, B_PER_W)
    tableT = embeddings_weight.T
    return _emb_lookup(idx2, tableT)


# speedup vs baseline: 2.9998x; 1.1670x over previous
import functools

import jax
import jax.numpy as jnp
from jax import lax
from jax.experimental import pallas as pl
from jax.experimental.pallas import tpu as pltpu
from jax.experimental.pallas import tpu_sc as plsc

V_DIM = 1000000
EMB_DIM = 64
BATCH = 16384

NC = 2
NS = 16
NW = NC * NS
B_PER_W = BATCH // NW
NBUF = 8
L = 16
NG = B_PER_W // L
FR = 128
GPF = FR // L


def _emb_kernel(idx_hbm, tableT_hbm, out_hbm, idx_v, cols_v, outS_v, sems,
                sem_out):
    wid = lax.axis_index("s") * NC + lax.axis_index("c")
    base = wid * B_PER_W
    pltpu.sync_copy(idx_hbm.at[wid], idx_v.at[pl.ds(0, B_PER_W)])

    def fire(x, slot):
        tc = lax.shift_right_logical(x, 7)
        off = pl.multiple_of(tc * 128, 128)
        pltpu.async_copy(
            tableT_hbm.at[:, pl.ds(off, 128)], cols_v.at[slot], sems.at[slot]
        )

    def wait(slot):
        pltpu.make_async_copy(
            tableT_hbm.at[:, pl.ds(0, 128)], cols_v.at[slot], sems.at[slot]
        ).wait()

    def wait_flush(half):
        pltpu.make_async_copy(
            outS_v.at[half], out_hbm.at[pl.ds(base, FR)], sem_out
        ).wait()

    def select(x, i, slot):
        c = jnp.broadcast_to(jnp.bitwise_and(x, 127), (L,))
        half = jnp.bitwise_and(lax.shift_right_logical(i, 7), 1)
        row = jnp.bitwise_and(i, FR - 1)
        for g in range(EMB_DIM // L):
            idx0 = lax.iota(jnp.int32, L) + g * L
            outS_v[half, row, pl.ds(g * L, L)] = plsc.load_gather(
                cols_v.at[slot], [idx0, c]
            )

    v0 = idx_v[pl.ds(0, L)]
    v1 = idx_v[pl.ds(L, L)]
    for b in range(NBUF):
        fire(v0[b] if b < L else v1[b - L], b)

    def body(g, _):
        @pl.when(jnp.bitwise_and(g, GPF - 1) == 0)
        def _():
            fl = lax.shift_right_logical(g, 3)

            @pl.when(fl >= 2)
            def _():
                wait_flush(jnp.bitwise_and(fl, 1))

        vcur = idx_v[pl.ds(g * L, L)]
        vnxt = idx_v[pl.ds(g * L + L, L)]
        for b in range(L):
            i = g * L + b
            slot = b % NBUF
            wait(slot)
            xn = vcur[b + NBUF] if b + NBUF < L else vnxt[b + NBUF - L]

            @pl.when(i < B_PER_W - NBUF)
            def _():
                fire(xn, slot)

            select(vcur[b], i, slot)

        @pl.when(jnp.bitwise_and(g, GPF - 1) == GPF - 1)
        def _():
            fl = lax.shift_right_logical(g, 3)
            half = jnp.bitwise_and(fl, 1)
            pltpu.async_copy(
                outS_v.at[half], out_hbm.at[pl.ds(base + fl * FR, FR)],
                sem_out,
            )

        return _

    lax.fori_loop(0, NG, body, None, unroll=False)
    wait_flush(0)
    wait_flush(1)


@jax.jit
def _emb_lookup(idx2, tableT):
    mesh = plsc.VectorSubcoreMesh(
        core_axis_name="c", subcore_axis_name="s", num_cores=NC, num_subcores=NS
    )
    return pl.kernel(
        _emb_kernel,
        out_type=jax.ShapeDtypeStruct((BATCH, EMB_DIM), jnp.float32),
        mesh=mesh,
        scratch_types=[
            pltpu.VMEM((B_PER_W + L,), jnp.int32),
            pltpu.VMEM((NBUF, EMB_DIM, 128), jnp.float32),
            pltpu.VMEM((2, FR, EMB_DIM), jnp.float32),
            pltpu.SemaphoreType.DMA((NBUF,)),
            pltpu.SemaphoreType.DMA,
        ],
        compiler_params=pltpu.CompilerParams(needs_layout_passes=False),
    )(idx2, tableT)


def kernel(x, embeddings_weight):
    idx2 = x.astype(jnp.int32).reshape(NW, B_PER_W)
    tableT = embeddings_weight.T
    return _emb_lookup(idx2, tableT)


# speedup vs baseline: 4.2818x; 1.4274x over previous
import functools

import jax
import jax.numpy as jnp
from jax import lax
from jax.experimental import pallas as pl
from jax.experimental.pallas import tpu as pltpu
from jax.experimental.pallas import tpu_sc as plsc

V_DIM = 1000000
EMB_DIM = 64
BATCH = 16384

NC = 2
NS = 16
NW = NC * NS
B_PER_W = BATCH // NW
NBUF = 11
D = 10
L = 16
NG = B_PER_W // L
PADW = B_PER_W + L
FR = 128
GPF = FR // L
KC = 128
NKC = B_PER_W // KC


def _gather_kernel(svp_hbm, flagp_hbm, slotp_hbm, tableT_hbm, out_hbm,
                   sv_v, fl_v, sl_v, cols_v, outS_v, sems, sem_out):
    wid = lax.axis_index("s") * NC + lax.axis_index("c")
    base = wid * B_PER_W
    pltpu.sync_copy(svp_hbm.at[wid], sv_v)
    pltpu.sync_copy(flagp_hbm.at[wid], fl_v)
    pltpu.sync_copy(slotp_hbm.at[wid], sl_v)

    def fire(x, slot):
        tc = lax.shift_right_logical(x, 7)
        off = pl.multiple_of(tc * 128, 128)
        pltpu.async_copy(
            tableT_hbm.at[:, pl.ds(off, 128)], cols_v.at[slot], sems.at[slot]
        )

    def wait(slot):
        pltpu.make_async_copy(
            tableT_hbm.at[:, pl.ds(0, 128)], cols_v.at[slot], sems.at[slot]
        ).wait()

    def wait_flush(half):
        pltpu.make_async_copy(
            outS_v.at[half], out_hbm.at[pl.ds(base, FR)], sem_out
        ).wait()

    def select(x, slot, i):
        c = jnp.broadcast_to(jnp.bitwise_and(x, 127), (L,))
        sl = jnp.broadcast_to(slot, (L,))
        half = jnp.bitwise_and(lax.shift_right_logical(i, 7), 1)
        row = jnp.bitwise_and(i, FR - 1)
        for g in range(EMB_DIM // L):
            idx0 = lax.iota(jnp.int32, L) + g * L
            outS_v[half, row, pl.ds(g * L, L)] = plsc.load_gather(
                cols_v, [sl, idx0, c]
            )

    sv0 = sv_v[pl.ds(0, L)]
    fl0 = fl_v[pl.ds(0, L)]
    sl0 = sl_v[pl.ds(0, L)]
    for b in range(D):

        @pl.when(fl0[b] == 1)
        def _():
            fire(sv0[b], sl0[b])

    def body(g, _):
        @pl.when(jnp.bitwise_and(g, GPF - 1) == 0)
        def _():
            fl = lax.shift_right_logical(g, 3)

            @pl.when(fl >= 2)
            def _():
                wait_flush(jnp.bitwise_and(fl, 1))

        svc = sv_v[pl.ds(g * L, L)]
        flc = fl_v[pl.ds(g * L, L)]
        slc = sl_v[pl.ds(g * L, L)]
        sva = sv_v[pl.ds(g * L + D, L)]
        fla = fl_v[pl.ds(g * L + D, L)]
        sla = sl_v[pl.ds(g * L + D, L)]
        for b in range(L):
            i = g * L + b

            @pl.when(fla[b] == 1)
            def _():
                fire(sva[b], sla[b])

            @pl.when(flc[b] == 1)
            def _():
                wait(slc[b])

            select(svc[b], slc[b], i)

        @pl.when(jnp.bitwise_and(g, GPF - 1) == GPF - 1)
        def _():
            fl = lax.shift_right_logical(g, 3)
            half = jnp.bitwise_and(fl, 1)
            pltpu.async_copy(
                outS_v.at[half], out_hbm.at[pl.ds(base + fl * FR, FR)],
                sem_out,
            )

        return _

    lax.fori_loop(0, NG, body, None, unroll=False)
    wait_flush(0)
    wait_flush(1)


def _perm_kernel(rank_hbm, rows_hbm, out_hbm, rk_v, rows_v, sem):
    wid = lax.axis_index("s") * NC + lax.axis_index("c")
    base = wid * B_PER_W
    pltpu.sync_copy(rank_hbm.at[wid], rk_v)
    copies = []
    for j in range(NKC):
        copies.append(
            pltpu.async_copy(
                rows_hbm.at[rk_v.at[j]],
                rows_v.at[pl.ds(j * KC, KC)],
                sem,
            )
        )
    for c in copies:
        c.wait()
    pltpu.sync_copy(rows_v, out_hbm.at[pl.ds(base, B_PER_W)])


@jax.jit
def _emb_lookup(svp, flagp, slotp, rank3, tableT):
    mesh = plsc.VectorSubcoreMesh(
        core_axis_name="c", subcore_axis_name="s", num_cores=NC, num_subcores=NS
    )
    sorted_rows = pl.kernel(
        _gather_kernel,
        out_type=jax.ShapeDtypeStruct((BATCH, EMB_DIM), jnp.float32),
        mesh=mesh,
        scratch_types=[
            pltpu.VMEM((PADW,), jnp.int32),
            pltpu.VMEM((PADW,), jnp.int32),
            pltpu.VMEM((PADW,), jnp.int32),
            pltpu.VMEM((NBUF, EMB_DIM, 128), jnp.float32),
            pltpu.VMEM((2, FR, EMB_DIM), jnp.float32),
            pltpu.SemaphoreType.DMA((NBUF,)),
            pltpu.SemaphoreType.DMA,
        ],
        compiler_params=pltpu.CompilerParams(needs_layout_passes=False),
    )(svp, flagp, slotp, tableT)
    return pl.kernel(
        _perm_kernel,
        out_type=jax.ShapeDtypeStruct((BATCH, EMB_DIM), jnp.float32),
        mesh=mesh,
        scratch_types=[
            pltpu.VMEM((NKC, KC), jnp.int32),
            pltpu.VMEM((B_PER_W, EMB_DIM), jnp.float32),
            pltpu.SemaphoreType.DMA,
        ],
        compiler_params=pltpu.CompilerParams(
            use_tc_tiling_on_sc=False, needs_layout_passes=False
        ),
    )(rank3, sorted_rows)


def kernel(x, embeddings_weight):
    xi = x.astype(jnp.int32)
    iota = lax.iota(jnp.int32, BATCH)
    sv, perm = lax.sort_key_val(xi, iota)
    rank = jnp.zeros((BATCH,), jnp.int32).at[perm].set(iota)
    sv2 = sv.reshape(NW, B_PER_W)
    tc2 = lax.shift_right_logical(sv2, 7)
    flag2 = jnp.concatenate(
        [
            jnp.ones((NW, 1), jnp.int32),
            (tc2[:, 1:] != tc2[:, :-1]).astype(jnp.int32),
        ],
        axis=1,
    )
    fo2 = jnp.cumsum(flag2, axis=1) - 1
    slot2 = jnp.remainder(fo2, NBUF)
    pad0 = jnp.zeros((NW, L), jnp.int32)
    svp = jnp.concatenate([sv2, jnp.tile(sv2[:, -1:], (1, L))], axis=1)
    flagp = jnp.concatenate([flag2, pad0], axis=1)
    slotp = jnp.concatenate([slot2, pad0], axis=1)
    rank3 = rank.reshape(NW, NKC, KC)
    tableT = embeddings_weight.T
    return _emb_lookup(svp, flagp, slotp, rank3, tableT)


# speedup vs baseline: 4.3550x; 1.0171x over previous
import functools

import jax
import jax.numpy as jnp
from jax import lax
from jax.experimental import pallas as pl
from jax.experimental.pallas import tpu as pltpu
from jax.experimental.pallas import tpu_sc as plsc

V_DIM = 1000000
EMB_DIM = 64
BATCH = 16384

NC = 2
NS = 16
NW = NC * NS
B_PER_W = BATCH // NW
NBUF = 12
D = 11
L = 16
NG = B_PER_W // L
PADW = B_PER_W + L
FR = 64
GPF = FR // L
NFL = B_PER_W // FR
KC = 128
NKC = B_PER_W // KC


def _gather_kernel(svp_hbm, flagp_hbm, slotp_hbm, tableT_hbm, out_hbm,
                   sv_v, fl_v, sl_v, cols_v, outS_v, sems, sem_out):
    wid = lax.axis_index("s") * NC + lax.axis_index("c")
    base = wid * B_PER_W
    pltpu.sync_copy(svp_hbm.at[wid], sv_v)
    pltpu.sync_copy(flagp_hbm.at[wid], fl_v)
    pltpu.sync_copy(slotp_hbm.at[wid], sl_v)

    def fire(x, slot):
        tc = lax.shift_right_logical(x, 7)
        off = pl.multiple_of(tc * 128, 128)
        pltpu.async_copy(
            tableT_hbm.at[:, pl.ds(off, 128)], cols_v.at[slot], sems.at[slot]
        )

    def wait(slot):
        pltpu.make_async_copy(
            tableT_hbm.at[:, pl.ds(0, 128)], cols_v.at[slot], sems.at[slot]
        ).wait()

    def wait_flush(half):
        pltpu.make_async_copy(
            outS_v.at[half], out_hbm.at[pl.ds(base, FR)], sem_out
        ).wait()

    def select(x, slot, i):
        c = jnp.broadcast_to(jnp.bitwise_and(x, 127), (L,))
        sl = jnp.broadcast_to(slot, (L,))
        half = jnp.bitwise_and(i // FR, 1)
        row = jnp.bitwise_and(i, FR - 1)
        for g in range(EMB_DIM // L):
            idx0 = lax.iota(jnp.int32, L) + g * L
            outS_v[half, row, pl.ds(g * L, L)] = plsc.load_gather(
                cols_v, [sl, idx0, c]
            )

    sv0 = sv_v[pl.ds(0, L)]
    fl0 = fl_v[pl.ds(0, L)]
    sl0 = sl_v[pl.ds(0, L)]
    for b in range(D):

        @pl.when(fl0[b] == 1)
        def _():
            fire(sv0[b], sl0[b])

    def body(g, _):
        @pl.when(jnp.bitwise_and(g, GPF - 1) == 0)
        def _():
            fl = g // GPF

            @pl.when(fl >= 2)
            def _():
                wait_flush(jnp.bitwise_and(fl, 1))

        svc = sv_v[pl.ds(g * L, L)]
        flc = fl_v[pl.ds(g * L, L)]
        slc = sl_v[pl.ds(g * L, L)]
        sva = sv_v[pl.ds(g * L + D, L)]
        fla = fl_v[pl.ds(g * L + D, L)]
        sla = sl_v[pl.ds(g * L + D, L)]
        for b in range(L):
            i = g * L + b

            @pl.when(fla[b] == 1)
            def _():
                fire(sva[b], sla[b])

            @pl.when(flc[b] == 1)
            def _():
                wait(slc[b])

            select(svc[b], slc[b], i)

        @pl.when(jnp.bitwise_and(g, GPF - 1) == GPF - 1)
        def _():
            fl = g // GPF
            half = jnp.bitwise_and(fl, 1)
            pltpu.async_copy(
                outS_v.at[half], out_hbm.at[pl.ds(base + fl * FR, FR)],
                sem_out,
            )

        return _

    lax.fori_loop(0, NG, body, None, unroll=False)
    wait_flush(0)
    wait_flush(1)


def _perm_kernel(perm_hbm, rows_hbm, out_hbm, pm_v, rows_v, sem):
    wid = lax.axis_index("s") * NC + lax.axis_index("c")
    base = wid * B_PER_W
    pltpu.sync_copy(perm_hbm.at[wid], pm_v)
    pltpu.sync_copy(rows_hbm.at[pl.ds(base, B_PER_W)], rows_v)
    copies = []
    for j in range(NKC):
        copies.append(
            pltpu.async_copy(
                rows_v.at[pl.ds(j * KC, KC)],
                out_hbm.at[pm_v.at[j]],
                sem,
            )
        )
    for c in copies:
        c.wait()


@jax.jit
def _emb_lookup(svp, flagp, slotp, perm3, tableT):
    mesh = plsc.VectorSubcoreMesh(
        core_axis_name="c", subcore_axis_name="s", num_cores=NC, num_subcores=NS
    )
    sorted_rows = pl.kernel(
        _gather_kernel,
        out_type=jax.ShapeDtypeStruct((BATCH, EMB_DIM), jnp.float32),
        mesh=mesh,
        scratch_types=[
            pltpu.VMEM((PADW,), jnp.int32),
            pltpu.VMEM((PADW,), jnp.int32),
            pltpu.VMEM((PADW,), jnp.int32),
            pltpu.VMEM((NBUF, EMB_DIM, 128), jnp.float32),
            pltpu.VMEM((2, FR, EMB_DIM), jnp.float32),
            pltpu.SemaphoreType.DMA((NBUF,)),
            pltpu.SemaphoreType.DMA,
        ],
        compiler_params=pltpu.CompilerParams(needs_layout_passes=False),
    )(svp, flagp, slotp, tableT)
    return pl.kernel(
        _perm_kernel,
        out_type=jax.ShapeDtypeStruct((BATCH, EMB_DIM), jnp.float32),
        mesh=mesh,
        scratch_types=[
            pltpu.VMEM((NKC, KC), jnp.int32),
            pltpu.VMEM((B_PER_W, EMB_DIM), jnp.float32),
            pltpu.SemaphoreType.DMA,
        ],
        compiler_params=pltpu.CompilerParams(
            use_tc_tiling_on_sc=False, needs_layout_passes=False
        ),
    )(perm3, sorted_rows)


def kernel(x, embeddings_weight):
    xi = x.astype(jnp.int32)
    iota = lax.iota(jnp.int32, BATCH)
    sv, perm = lax.sort_key_val(xi, iota)
    sv2 = sv.reshape(NW, B_PER_W)
    tc2 = lax.shift_right_logical(sv2, 7)
    flag2 = jnp.concatenate(
        [
            jnp.ones((NW, 1), jnp.int32),
            (tc2[:, 1:] != tc2[:, :-1]).astype(jnp.int32),
        ],
        axis=1,
    )
    fo2 = jnp.cumsum(flag2, axis=1) - 1
    slot2 = jnp.remainder(fo2, NBUF)
    pad0 = jnp.zeros((NW, L), jnp.int32)
    svp = jnp.concatenate([sv2, jnp.tile(sv2[:, -1:], (1, L))], axis=1)
    flagp = jnp.concatenate([flag2, pad0], axis=1)
    slotp = jnp.concatenate([slot2, pad0], axis=1)
    perm3 = perm.reshape(NW, NKC, KC)
    tableT = embeddings_weight.T
    return _emb_lookup(svp, flagp, slotp, perm3, tableT)
